# Initial kernel scaffold; baseline (speedup 1.0000x reference)
import jax
import jax.numpy as jnp
from jax.experimental import pallas as pl


def kernel(v1s, v2s, cmaps):
    raise NotImplementedError("write your pallas kernel here")


# speedup vs baseline: 7.9277x; 7.9277x over previous
import functools

import jax
import jax.numpy as jnp
from jax.experimental import pallas as pl


def _cmap_min_dist_kernel(v1_ref, v2_ref, cm_ref, out_ref):
    v1 = v1_ref[0]
    v2 = v2_ref[0]
    cm = cm_ref[0]

    n, _ = v1.shape
    r = cm.shape[0]
    k = n // r

    n1c = jnp.sum(v1 * v1, axis=1, keepdims=True)
    n2r = jax.lax.dot_general(
        jnp.ones((1, v2.shape[1]), jnp.float32), v2 * v2,
        (((1,), (1,)), ((), ())),
        precision=jax.lax.Precision.HIGHEST,
        preferred_element_type=jnp.float32)

    g = jax.lax.dot_general(
        v1, v2, (((1,), (1,)), ((), ())),
        preferred_element_type=jnp.float32)
    h = (n1c + n2r) - 2.0 * g

    rows = [jnp.min(h[i * k:(i + 1) * k, :], axis=0, keepdims=True)
            for i in range(r)]
    s1 = jnp.concatenate(rows, axis=0)

    cols = [jnp.min(s1[:, j * k:(j + 1) * k], axis=1, keepdims=True)
            for j in range(r)]
    md2 = jnp.concatenate(cols, axis=1)

    d = jnp.sqrt(jnp.maximum(md2, 1e-12))
    denom = jnp.maximum(jnp.sum(cm), 1.0)
    val = jnp.sum(d * cm) / denom
    out_ref[...] = jnp.broadcast_to(val, out_ref.shape)


@jax.jit
def kernel(v1s, v2s, cmaps):
    b, n, _ = v1s.shape
    r = cmaps.shape[1]
    cm = cmaps.astype(jnp.float32)
    out = pl.pallas_call(
        _cmap_min_dist_kernel,
        grid=(b,),
        in_specs=[
            pl.BlockSpec((1, n, v1s.shape[2]), lambda i: (i, 0, 0)),
            pl.BlockSpec((1, n, v2s.shape[2]), lambda i: (i, 0, 0)),
            pl.BlockSpec((1, r, r), lambda i: (i, 0, 0)),
        ],
        out_specs=pl.BlockSpec((1, 1, 128), lambda i: (i, 0, 0)),
        out_shape=jax.ShapeDtypeStruct((b, 1, 128), jnp.float32),
    )(v1s, v2s, cm)
    return out[:, 0, 0]


# speedup vs baseline: 8.4662x; 1.0679x over previous
import functools

import jax
import jax.numpy as jnp
from jax.experimental import pallas as pl
from jax.experimental.pallas import tpu as pltpu


def _cmap_min_dist_kernel(v1_ref, v2_ref, cm_ref, out_ref):
    v1 = v1_ref[0]
    v2 = v2_ref[0]
    cm = cm_ref[0]

    n, _ = v1.shape
    r = cm.shape[0]
    k = n // r

    n1c = jnp.sum(v1 * v1, axis=1, keepdims=True)
    n2r = jax.lax.dot_general(
        jnp.ones((1, v2.shape[1]), jnp.float32), v2 * v2,
        (((1,), (1,)), ((), ())),
        precision=jax.lax.Precision.HIGHEST,
        preferred_element_type=jnp.float32)

    g2 = jax.lax.dot_general(
        -2.0 * v1, v2, (((1,), (1,)), ((), ())),
        preferred_element_type=jnp.float32)
    h = (n1c + n2r) + g2

    rows = [jnp.min(h[i * k:(i + 1) * k, :], axis=0, keepdims=True)
            for i in range(r)]
    s1 = jnp.concatenate(rows, axis=0)

    cols = [jnp.min(s1[:, j * k:(j + 1) * k], axis=1, keepdims=True)
            for j in range(r)]
    md2 = jnp.concatenate(cols, axis=1)

    d = jnp.sqrt(jnp.maximum(md2, 1e-12))
    denom = jnp.maximum(jnp.sum(cm), 1.0)
    val = jnp.sum(d * cm) / denom
    out_ref[...] = jnp.broadcast_to(val, out_ref.shape)


@jax.jit
def kernel(v1s, v2s, cmaps):
    b, n, _ = v1s.shape
    r = cmaps.shape[1]
    cm = cmaps.astype(jnp.float32)
    out = pl.pallas_call(
        _cmap_min_dist_kernel,
        grid=(b,),
        in_specs=[
            pl.BlockSpec((1, n, v1s.shape[2]), lambda i: (i, 0, 0)),
            pl.BlockSpec((1, n, v2s.shape[2]), lambda i: (i, 0, 0)),
            pl.BlockSpec((1, r, r), lambda i: (i, 0, 0)),
        ],
        out_specs=pl.BlockSpec((1, 1, 128), lambda i: (i, 0, 0)),
        out_shape=jax.ShapeDtypeStruct((b, 1, 128), jnp.float32),
        compiler_params=pltpu.CompilerParams(
            dimension_semantics=("parallel",)),
    )(v1s, v2s, cm)
    return out[:, 0, 0]


# speedup vs baseline: 8.9040x; 1.0517x over previous
import functools

import jax
import jax.numpy as jnp
from jax.experimental import pallas as pl
from jax.experimental.pallas import tpu as pltpu


def _cmap_min_dist_kernel(v1_ref, v2_ref, cm_ref, out_ref):
    v1 = v1_ref[0]
    v2 = v2_ref[0]
    cm = cm_ref[0]

    n, _ = v1.shape
    r = cm.shape[0]
    k = n // r

    n1c = jnp.sum(v1 * v1, axis=1, keepdims=True)
    n2c = jnp.sum(v2 * v2, axis=1, keepdims=True)

    def _bf16_split3(x):
        hi = x.astype(jnp.bfloat16).astype(jnp.float32)
        rem = x - hi
        mid = rem.astype(jnp.bfloat16).astype(jnp.float32)
        return hi, mid, rem - mid

    ones = jnp.ones_like(n1c)
    h1, m1, l1 = _bf16_split3(n1c)
    h2, m2, l2 = _bf16_split3(n2c)
    v1a = jnp.concatenate([-2.0 * v1, h1, m1, l1, ones, ones, ones], axis=1)
    v2a = jnp.concatenate([v2, ones, ones, ones, h2, m2, l2], axis=1)
    h = jax.lax.dot_general(
        v1a, v2a, (((1,), (1,)), ((), ())),
        preferred_element_type=jnp.float32)

    rows = [jnp.min(h[i * k:(i + 1) * k, :], axis=0, keepdims=True)
            for i in range(r)]
    s1 = jnp.concatenate(rows, axis=0)

    cols = [jnp.min(s1[:, j * k:(j + 1) * k], axis=1, keepdims=True)
            for j in range(r)]
    md2 = jnp.concatenate(cols, axis=1)

    d = jnp.sqrt(jnp.maximum(md2, 1e-12))
    denom = jnp.maximum(jnp.sum(cm), 1.0)
    val = jnp.sum(d * cm) / denom
    out_ref[...] = jnp.broadcast_to(val, out_ref.shape)


@jax.jit
def kernel(v1s, v2s, cmaps):
    b, n, _ = v1s.shape
    r = cmaps.shape[1]
    cm = cmaps.astype(jnp.float32)
    out = pl.pallas_call(
        _cmap_min_dist_kernel,
        grid=(b,),
        in_specs=[
            pl.BlockSpec((1, n, v1s.shape[2]), lambda i: (i, 0, 0)),
            pl.BlockSpec((1, n, v2s.shape[2]), lambda i: (i, 0, 0)),
            pl.BlockSpec((1, r, r), lambda i: (i, 0, 0)),
        ],
        out_specs=pl.BlockSpec((1, 1, 128), lambda i: (i, 0, 0)),
        out_shape=jax.ShapeDtypeStruct((b, 1, 128), jnp.float32),
        compiler_params=pltpu.CompilerParams(
            dimension_semantics=("parallel",)),
    )(v1s, v2s, cm)
    return out[:, 0, 0]
